# baseline (speedup 1.0000x reference)
import functools

import jax
import jax.numpy as jnp
from jax.experimental import pallas as pl

E = 8
H = 32
P = E * H


def _fwd_kernel(x_ref, w1_ref, b1_ref, w2_ref, b2_ref, w3_ref, b3_ref, o_ref):
    xb = x_ref[...]
    idx_f = (xb + 1.0) * 0.5
    idx_f = idx_f * 2.0
    ii = jnp.clip(idx_f.astype(jnp.int32), 0, 1)
    mid = ii[:, 0:1] + 2 * ii[:, 1:2] + 4 * ii[:, 2:3]
    eids = jax.lax.broadcasted_iota(jnp.int32, (xb.shape[0], E), 1)
    onehot = (mid == eids).astype(jnp.float32)

    h = jnp.tanh(
        jnp.dot(xb, w1_ref[...], preferred_element_type=jnp.float32) + b1_ref[...]
    )
    h = jnp.tanh(
        jnp.dot(h, w2_ref[...], preferred_element_type=jnp.float32) + b2_ref[...]
    )
    y8 = jnp.dot(h, w3_ref[...], preferred_element_type=jnp.float32) + b3_ref[...]
    o_ref[...] = jnp.sum(y8 * onehot, axis=1, keepdims=True)


@functools.partial(jax.jit, static_argnames=())
def kernel(x, W1, b1, W2, b2, W3, b3):
    n = x.shape[0]
    w1c = jnp.transpose(W1, (1, 0, 2)).reshape(3, P)
    b1c = b1.reshape(1, P)
    w2bd = jnp.zeros((P, P), jnp.float32)
    w3bc = jnp.zeros((P, E), jnp.float32)
    for e in range(E):
        w2bd = jax.lax.dynamic_update_slice(w2bd, W2[e], (e * H, e * H))
        w3bc = jax.lax.dynamic_update_slice(w3bc, W3[e], (e * H, e))
    b2c = b2.reshape(1, P)
    b3r = b3.reshape(1, E)

    B = 2048
    grid = (n // B,)
    out = pl.pallas_call(
        _fwd_kernel,
        grid=grid,
        in_specs=[
            pl.BlockSpec((B, 3), lambda i: (i, 0)),
            pl.BlockSpec((3, P), lambda i: (0, 0)),
            pl.BlockSpec((1, P), lambda i: (0, 0)),
            pl.BlockSpec((P, P), lambda i: (0, 0)),
            pl.BlockSpec((1, P), lambda i: (0, 0)),
            pl.BlockSpec((P, E), lambda i: (0, 0)),
            pl.BlockSpec((1, E), lambda i: (0, 0)),
        ],
        out_specs=pl.BlockSpec((B, 1), lambda i: (i, 0)),
        out_shape=jax.ShapeDtypeStruct((n, 1), jnp.float32),
    )(x, w1c, b1c, w2bd, b2c, w3bc, b3r)
    return out


# speedup vs baseline: 1.2119x; 1.2119x over previous
import jax
import jax.numpy as jnp
from jax.experimental import pallas as pl

E = 8
H = 32
P = E * H

_THRESH = -(2.0 ** -25)


def _fwd_kernel(x_ref, w1_ref, b1_ref, w2_ref, b2_ref, w3_ref, b3_ref, o_ref):
    xb = x_ref[...]
    bits = jnp.where(xb >= _THRESH, 1.0, 0.0).astype(jnp.bfloat16)
    dsel = jax.lax.broadcasted_iota(jnp.int32, (3, E), 0)
    wsel8 = (1 << dsel).astype(jnp.bfloat16)
    mid8 = jax.lax.dot_general(
        bits, wsel8, (((1,), (0,)), ((), ())),
        preferred_element_type=jnp.float32,
    )
    mid8i = mid8.astype(jnp.int32)
    eids = jax.lax.broadcasted_iota(jnp.int32, mid8.shape, 1)

    xbf = xb.astype(jnp.bfloat16)
    a1 = (
        jax.lax.dot_general(
            xbf, w1_ref[...], (((1,), (0,)), ((), ())),
            preferred_element_type=jnp.float32,
        )
        + b1_ref[...]
    )
    t1 = jnp.tanh(a1).astype(jnp.bfloat16)
    a2 = (
        jax.lax.dot_general(
            t1, w2_ref[...], (((1,), (0,)), ((), ())),
            preferred_element_type=jnp.float32,
        )
        + b2_ref[...]
    )
    t2 = jnp.tanh(a2).astype(jnp.bfloat16)
    y8 = (
        jax.lax.dot_general(
            t2, w3_ref[...], (((1,), (0,)), ((), ())),
            preferred_element_type=jnp.float32,
        )
        + b3_ref[...]
    )
    ysel = jnp.where(mid8i == eids, y8, 0.0)
    o_ref[...] = jnp.sum(ysel, axis=1, keepdims=True)


def kernel(x, W1, b1, W2, b2, W3, b3):
    n = x.shape[0]
    w1c = jnp.transpose(W1, (1, 0, 2)).reshape(3, P).astype(jnp.bfloat16)
    b1c = b1.reshape(1, P)
    eye = jnp.eye(E, dtype=jnp.float32)
    w2bd = (eye[:, None, :, None] * W2[:, :, None, :]).reshape(P, P).astype(jnp.bfloat16)
    b2c = b2.reshape(1, P)
    w3bc = (eye[:, None, :] * W3.reshape(E, H, 1)).reshape(P, E).astype(jnp.bfloat16)
    b3r = b3.reshape(1, E)

    B = 4096
    grid = (n // B,)
    out = pl.pallas_call(
        _fwd_kernel,
        grid=grid,
        in_specs=[
            pl.BlockSpec((B, 3), lambda i: (i, 0)),
            pl.BlockSpec((3, P), lambda i: (0, 0)),
            pl.BlockSpec((1, P), lambda i: (0, 0)),
            pl.BlockSpec((P, P), lambda i: (0, 0)),
            pl.BlockSpec((1, P), lambda i: (0, 0)),
            pl.BlockSpec((P, E), lambda i: (0, 0)),
            pl.BlockSpec((1, E), lambda i: (0, 0)),
        ],
        out_specs=pl.BlockSpec((B, 1), lambda i: (i, 0)),
        out_shape=jax.ShapeDtypeStruct((n, 1), jnp.float32),
    )(x, w1c, b1c, w2bd, b2c, w3bc, b3r)
    return out


# speedup vs baseline: 1.2274x; 1.0128x over previous
import jax
import jax.numpy as jnp
from jax.experimental import pallas as pl
from jax.experimental.pallas import tpu as pltpu

E = 8
H = 32
DIN = 3
P = E * H

_THRESH = -(2.0 ** -25)


def _fwd_kernel(
    x_ref, w1_ref, b1_ref, w2_ref, b2_ref, w3_ref, b3_ref, o_ref,
    w1c_s, w2bd_s, w3bc_s, b1c_s, b2c_s, b3r_s,
):
    bf16 = jnp.bfloat16
    f32 = jnp.float32

    @pl.when(pl.program_id(0) == 0)
    def _init():
        w1 = w1_ref[...]
        w1c_s[...] = jnp.concatenate(
            [w1[e] for e in range(E)], axis=1
        ).astype(bf16)
        w2 = w2_ref[...]
        w2flat = jnp.concatenate([w2[e] for e in range(E)], axis=0)
        w2tile = jnp.concatenate([w2flat] * E, axis=1)
        rowg = jax.lax.broadcasted_iota(jnp.int32, (P, P), 0) // H
        colg = jax.lax.broadcasted_iota(jnp.int32, (P, P), 1) // H
        w2bd_s[...] = jnp.where(rowg == colg, w2tile, 0.0).astype(bf16)
        w3 = w3_ref[...]
        w3flat = jnp.concatenate([w3[e] for e in range(E)], axis=0)
        w3tile = jnp.concatenate([w3flat] * E, axis=1)
        rowg3 = jax.lax.broadcasted_iota(jnp.int32, (P, E), 0) // H
        colg3 = jax.lax.broadcasted_iota(jnp.int32, (P, E), 1)
        w3bc_s[...] = jnp.where(rowg3 == colg3, w3tile, 0.0).astype(bf16)
        b1 = b1_ref[...]
        b1c_s[...] = jnp.concatenate([b1[e : e + 1, :] for e in range(E)], axis=1)
        b2 = b2_ref[...]
        b2c_s[...] = jnp.concatenate([b2[e : e + 1, :] for e in range(E)], axis=1)
        b3 = b3_ref[...]
        b3r_s[...] = jnp.concatenate([b3[e : e + 1, :] for e in range(E)], axis=1)

    xb = x_ref[...]
    bits = jnp.where(xb >= _THRESH, 1.0, 0.0).astype(bf16)
    dsel = jax.lax.broadcasted_iota(jnp.int32, (DIN, E), 0)
    wsel8 = (1 << dsel).astype(bf16)
    mid8 = jax.lax.dot_general(
        bits, wsel8, (((1,), (0,)), ((), ())), preferred_element_type=f32
    )
    mid8i = mid8.astype(jnp.int32)
    eids = jax.lax.broadcasted_iota(jnp.int32, mid8.shape, 1)

    xbf = xb.astype(bf16)
    a1 = (
        jax.lax.dot_general(
            xbf, w1c_s[...], (((1,), (0,)), ((), ())), preferred_element_type=f32
        )
        + b1c_s[...]
    )
    t1 = jnp.tanh(a1).astype(bf16)
    a2 = (
        jax.lax.dot_general(
            t1, w2bd_s[...], (((1,), (0,)), ((), ())), preferred_element_type=f32
        )
        + b2c_s[...]
    )
    t2 = jnp.tanh(a2).astype(bf16)
    y8 = (
        jax.lax.dot_general(
            t2, w3bc_s[...], (((1,), (0,)), ((), ())), preferred_element_type=f32
        )
        + b3r_s[...]
    )
    ysel = jnp.where(mid8i == eids, y8, 0.0)
    o_ref[...] = jnp.sum(ysel, axis=1, keepdims=True)


def kernel(x, W1, b1, W2, b2, W3, b3):
    n = x.shape[0]
    B = 4096
    grid = (n // B,)
    out = pl.pallas_call(
        _fwd_kernel,
        grid=grid,
        in_specs=[
            pl.BlockSpec((B, DIN), lambda i: (i, 0)),
            pl.BlockSpec((E, DIN, H), lambda i: (0, 0, 0)),
            pl.BlockSpec((E, H), lambda i: (0, 0)),
            pl.BlockSpec((E, H, H), lambda i: (0, 0, 0)),
            pl.BlockSpec((E, H), lambda i: (0, 0)),
            pl.BlockSpec((E, H, 1), lambda i: (0, 0, 0)),
            pl.BlockSpec((E, 1), lambda i: (0, 0)),
        ],
        out_specs=pl.BlockSpec((B, 1), lambda i: (i, 0)),
        out_shape=jax.ShapeDtypeStruct((n, 1), jnp.float32),
        scratch_shapes=[
            pltpu.VMEM((DIN, P), jnp.bfloat16),
            pltpu.VMEM((P, P), jnp.bfloat16),
            pltpu.VMEM((P, E), jnp.bfloat16),
            pltpu.VMEM((1, P), jnp.float32),
            pltpu.VMEM((1, P), jnp.float32),
            pltpu.VMEM((1, E), jnp.float32),
        ],
    )(x, W1, b1, W2, b2, W3, b3)
    return out


# speedup vs baseline: 3.1357x; 2.5547x over previous
import jax
import jax.numpy as jnp
from jax.experimental import pallas as pl
from jax.experimental.pallas import tpu as pltpu

E = 8
H = 32
DIN = 3
P = E * H

_THRESH = -(2.0 ** -25)


def _fwd_kernel(
    xt_ref, w1_ref, b1_ref, w2_ref, b2_ref, w3_ref, b3_ref, o_ref,
    w1t_s, w2t_s, w3t_s, b1t_s, b2t_s,
):
    bf16 = jnp.bfloat16
    f32 = jnp.float32

    @pl.when(pl.program_id(0) == 0)
    def _init():
        w1 = w1_ref[...]
        w1t_s[...] = jnp.concatenate(
            [jnp.transpose(w1[e]) for e in range(E)], axis=0
        ).astype(bf16)
        w2 = w2_ref[...]
        w2tf = jnp.concatenate(
            [jnp.transpose(w2[e]) for e in range(E)], axis=0
        )
        w2tile = jnp.concatenate([w2tf] * E, axis=1)
        rowg = jax.lax.broadcasted_iota(jnp.int32, (P, P), 0) // H
        colg = jax.lax.broadcasted_iota(jnp.int32, (P, P), 1) // H
        w2t_s[...] = jnp.where(rowg == colg, w2tile, 0.0).astype(bf16)
        w3 = w3_ref[...]
        w3row = jnp.concatenate(
            [jnp.transpose(w3[e]) for e in range(E)], axis=1
        )
        rowe = jax.lax.broadcasted_iota(jnp.int32, (E, P), 0)
        cole = jax.lax.broadcasted_iota(jnp.int32, (E, P), 1) // H
        w3t_s[...] = jnp.where(
            rowe == cole, jnp.broadcast_to(w3row, (E, P)), 0.0
        ).astype(bf16)
        b1 = b1_ref[...]
        b1t_s[...] = jnp.concatenate(
            [jnp.transpose(b1[e : e + 1, :]) for e in range(E)], axis=0
        )
        b2 = b2_ref[...]
        b2t_s[...] = jnp.concatenate(
            [jnp.transpose(b2[e : e + 1, :]) for e in range(E)], axis=0
        )

    xt = xt_ref[...]
    B = xt.shape[1]
    bits = jnp.where(xt >= _THRESH, 1.0, 0.0)
    mid = bits[0:1, :] + 2.0 * bits[1:2, :] + 4.0 * bits[2:3, :]
    midi = jnp.broadcast_to(mid.astype(jnp.int32), (E, B))
    eids = jax.lax.broadcasted_iota(jnp.int32, (E, B), 0)

    xbf = xt.astype(bf16)
    a1 = (
        jax.lax.dot_general(
            w1t_s[...], xbf, (((1,), (0,)), ((), ())), preferred_element_type=f32
        )
        + b1t_s[...]
    )
    t1 = jnp.tanh(a1).astype(bf16)
    a2 = (
        jax.lax.dot_general(
            w2t_s[...], t1, (((1,), (0,)), ((), ())), preferred_element_type=f32
        )
        + b2t_s[...]
    )
    t2 = jnp.tanh(a2).astype(bf16)
    y8 = (
        jax.lax.dot_general(
            w3t_s[...], t2, (((1,), (0,)), ((), ())), preferred_element_type=f32
        )
        + b3_ref[...]
    )
    ysel = jnp.where(midi == eids, y8, 0.0)
    o_ref[...] = jnp.sum(ysel, axis=0, keepdims=True)[None]


def kernel(x, W1, b1, W2, b2, W3, b3):
    n = x.shape[0]
    B = 4096
    nb = n // B
    xt = jnp.transpose(x)
    out = pl.pallas_call(
        _fwd_kernel,
        grid=(nb,),
        in_specs=[
            pl.BlockSpec((DIN, B), lambda i: (0, i)),
            pl.BlockSpec((E, DIN, H), lambda i: (0, 0, 0)),
            pl.BlockSpec((E, H), lambda i: (0, 0)),
            pl.BlockSpec((E, H, H), lambda i: (0, 0, 0)),
            pl.BlockSpec((E, H), lambda i: (0, 0)),
            pl.BlockSpec((E, H, 1), lambda i: (0, 0, 0)),
            pl.BlockSpec((E, 1), lambda i: (0, 0)),
        ],
        out_specs=pl.BlockSpec((1, 1, B), lambda i: (i, 0, 0)),
        out_shape=jax.ShapeDtypeStruct((nb, 1, B), jnp.float32),
        scratch_shapes=[
            pltpu.VMEM((P, DIN), jnp.bfloat16),
            pltpu.VMEM((P, P), jnp.bfloat16),
            pltpu.VMEM((E, P), jnp.bfloat16),
            pltpu.VMEM((P, 1), jnp.float32),
            pltpu.VMEM((P, 1), jnp.float32),
        ],
    )(xt, W1, b1, W2, b2, W3, b3)
    return out.reshape(n, 1)


# speedup vs baseline: 5.5105x; 1.7574x over previous
import jax
import jax.numpy as jnp
from jax.experimental import pallas as pl
from jax.experimental.pallas import tpu as pltpu

E = 8
H = 32
DIN = 3
P = E * H
K2 = P + E

_THRESH = -(2.0 ** -25)


def _fwd_kernel(
    xt_ref, w1_ref, b1_ref, w2_ref, b2_ref, w3_ref, b3_ref, o_ref,
    tab1_s, tab2_s,
):
    bf16 = jnp.bfloat16
    f32 = jnp.float32

    @pl.when(pl.program_id(0) == 0)
    def _init():
        w1 = w1_ref[...]
        tab1_s[...] = jnp.concatenate(
            [jnp.transpose(w1[:, d, :]) for d in range(DIN)]
            + [jnp.transpose(b1_ref[...])],
            axis=1,
        ).astype(bf16)
        w2 = w2_ref[...]
        tab2_s[...] = jnp.concatenate(
            [jnp.transpose(w2[e]) for e in range(E)]
            + [jnp.transpose(b2_ref[...])],
            axis=1,
        ).astype(bf16)

    xt = xt_ref[...]
    B = xt.shape[1]
    bits = jnp.where(xt >= _THRESH, 1.0, 0.0)
    mid = bits[0:1, :] + 2.0 * bits[1:2, :] + 4.0 * bits[2:3, :]
    midi = mid.astype(jnp.int32)
    mid8 = jnp.broadcast_to(midi, (E, B))
    eids8 = jax.lax.broadcasted_iota(jnp.int32, (E, B), 0)
    oh8 = jnp.where(mid8 == eids8, 1.0, 0.0)

    x24 = jnp.concatenate(
        [jnp.broadcast_to(xt[d : d + 1, :], (E, B)) for d in range(DIN)], axis=0
    )
    oh24 = jnp.concatenate([oh8] * DIN, axis=0)
    xk = jnp.concatenate([x24 * oh24, oh8], axis=0).astype(bf16)
    a1 = jax.lax.dot_general(
        tab1_s[...], xk, (((1,), (0,)), ((), ())), preferred_element_type=f32
    )
    t1 = jnp.tanh(a1).astype(bf16)

    t1rep = jnp.concatenate([t1] * E, axis=0)
    mid256 = jnp.broadcast_to(midi, (P, B))
    rows256 = jax.lax.broadcasted_iota(jnp.int32, (P, B), 0) // H
    featm = jnp.where(rows256 == mid256, t1rep, bf16(0))
    feat = jnp.concatenate([featm, oh8.astype(bf16)], axis=0)
    a2 = jax.lax.dot_general(
        tab2_s[...], feat, (((1,), (0,)), ((), ())), preferred_element_type=f32
    )
    t2 = jnp.tanh(a2).astype(bf16)

    w3flat = w3_ref[...].reshape(E, H).astype(bf16)
    y8 = (
        jax.lax.dot_general(
            w3flat, t2, (((1,), (0,)), ((), ())), preferred_element_type=f32
        )
        + b3_ref[...]
    )
    ysel = jnp.where(mid8 == eids8, y8, 0.0)
    o_ref[...] = jnp.sum(ysel, axis=0, keepdims=True)[None]


def kernel(x, W1, b1, W2, b2, W3, b3):
    n = x.shape[0]
    B = 4096
    nb = n // B
    xt = jnp.transpose(x)
    out = pl.pallas_call(
        _fwd_kernel,
        grid=(nb,),
        in_specs=[
            pl.BlockSpec((DIN, B), lambda i: (0, i)),
            pl.BlockSpec((E, DIN, H), lambda i: (0, 0, 0)),
            pl.BlockSpec((E, H), lambda i: (0, 0)),
            pl.BlockSpec((E, H, H), lambda i: (0, 0, 0)),
            pl.BlockSpec((E, H), lambda i: (0, 0)),
            pl.BlockSpec((E, H, 1), lambda i: (0, 0, 0)),
            pl.BlockSpec((E, 1), lambda i: (0, 0)),
        ],
        out_specs=pl.BlockSpec((1, 1, B), lambda i: (i, 0, 0)),
        out_shape=jax.ShapeDtypeStruct((nb, 1, B), jnp.float32),
        scratch_shapes=[
            pltpu.VMEM((H, H), jnp.bfloat16),
            pltpu.VMEM((H, K2), jnp.bfloat16),
        ],
    )(xt, W1, b1, W2, b2, W3, b3)
    return out.reshape(n, 1)


# speedup vs baseline: 7.4275x; 1.3479x over previous
import jax
import jax.numpy as jnp
from jax.experimental import pallas as pl
from jax.experimental.pallas import tpu as pltpu

E = 8
H = 32
DIN = 3
P = E * H
K2 = P + E

_THRESH = -(2.0 ** -25)


def _fwd_kernel(
    xt_ref, w1_ref, b1_ref, w2_ref, b2_ref, w3_ref, b3_ref, o_ref,
    tab1_s, tab2_s,
):
    bf16 = jnp.bfloat16
    f32 = jnp.float32

    @pl.when(pl.program_id(0) == 0)
    def _init():
        w1 = w1_ref[...]
        tab1_s[...] = jnp.concatenate(
            [jnp.transpose(w1[:, d, :]) for d in range(DIN)]
            + [jnp.transpose(b1_ref[...])],
            axis=1,
        ).astype(bf16)
        w2 = w2_ref[...]
        tab2_s[...] = jnp.concatenate(
            [jnp.transpose(w2[e]) for e in range(E)]
            + [jnp.transpose(b2_ref[...])],
            axis=1,
        ).astype(bf16)

    xt = xt_ref[...]
    B = xt.shape[1]
    bits = jnp.where(xt >= _THRESH, 1.0, 0.0)
    mid = bits[0:1, :] + 2.0 * bits[1:2, :] + 4.0 * bits[2:3, :]
    midi = mid.astype(jnp.int32)
    mid8 = jnp.broadcast_to(midi, (E, B))
    eids8 = jax.lax.broadcasted_iota(jnp.int32, (E, B), 0)
    oh8 = jnp.where(mid8 == eids8, 1.0, 0.0)

    x24 = jnp.concatenate(
        [jnp.broadcast_to(xt[d : d + 1, :], (E, B)) for d in range(DIN)], axis=0
    )
    oh24 = jnp.concatenate([oh8] * DIN, axis=0)
    xk = jnp.concatenate([x24 * oh24, oh8], axis=0).astype(bf16)
    a1 = jax.lax.dot_general(
        tab1_s[...], xk, (((1,), (0,)), ((), ())), preferred_element_type=f32
    )
    t1 = jnp.tanh(a1).astype(bf16)

    oh8b = oh8.astype(bf16)
    t1rep = jnp.broadcast_to(t1[None], (E, H, B)).reshape(P, B)
    oh256 = jnp.broadcast_to(oh8b[:, None, :], (E, H, B)).reshape(P, B)
    featm = t1rep * oh256
    feat = jnp.concatenate([featm, oh8b], axis=0)
    a2 = jax.lax.dot_general(
        tab2_s[...], feat, (((1,), (0,)), ((), ())), preferred_element_type=f32
    )
    t2 = jnp.tanh(a2).astype(bf16)

    w3flat = w3_ref[...].reshape(E, H).astype(bf16)
    y8 = (
        jax.lax.dot_general(
            w3flat, t2, (((1,), (0,)), ((), ())), preferred_element_type=f32
        )
        + b3_ref[...]
    )
    ysel = jnp.where(mid8 == eids8, y8, 0.0)
    o_ref[...] = jnp.sum(ysel, axis=0, keepdims=True)[None]


def kernel(x, W1, b1, W2, b2, W3, b3):
    n = x.shape[0]
    B = 16384
    nb = n // B
    xt = jnp.transpose(x)
    out = pl.pallas_call(
        _fwd_kernel,
        grid=(nb,),
        in_specs=[
            pl.BlockSpec((DIN, B), lambda i: (0, i)),
            pl.BlockSpec((E, DIN, H), lambda i: (0, 0, 0)),
            pl.BlockSpec((E, H), lambda i: (0, 0)),
            pl.BlockSpec((E, H, H), lambda i: (0, 0, 0)),
            pl.BlockSpec((E, H), lambda i: (0, 0)),
            pl.BlockSpec((E, H, 1), lambda i: (0, 0, 0)),
            pl.BlockSpec((E, 1), lambda i: (0, 0)),
        ],
        out_specs=pl.BlockSpec((1, 1, B), lambda i: (i, 0, 0)),
        out_shape=jax.ShapeDtypeStruct((nb, 1, B), jnp.float32),
        scratch_shapes=[
            pltpu.VMEM((H, H), jnp.bfloat16),
            pltpu.VMEM((H, K2), jnp.bfloat16),
        ],
    )(xt, W1, b1, W2, b2, W3, b3)
    return out.reshape(n, 1)
